# baseline (speedup 1.0000x reference)
import jax
import jax.numpy as jnp
from jax.experimental import pallas as pl
from jax.experimental.pallas import tpu as pltpu

_TILE_M = 1024


_FF_CHUNKS = 4


def _ffn_body(x_ref, w1_ref, b1_ref, w2_ref, b2_ref, o_ref):
    ff = w1_ref.shape[1]
    fc = ff // _FF_CHUNKS
    xb = x_ref[...].astype(jnp.bfloat16)
    acc = None
    for j in range(_FF_CHUNKS):
        sl = pl.ds(j * fc, fc)
        h = jnp.dot(xb, w1_ref[:, sl], preferred_element_type=jnp.float32)
        h = jnp.maximum(h + b1_ref[:, sl], 0.0).astype(jnp.bfloat16)
        p = jnp.dot(h, w2_ref[sl, :], preferred_element_type=jnp.float32)
        acc = p if acc is None else acc + p
    o_ref[...] = (acc + b2_ref[...]).astype(o_ref.dtype)


def _ffn_call(m_rows, tile_m, d_in, d_mid, d_out, out_dtype):
    const = lambda i: (0, 0)
    wkw = {"pipeline_mode": pl.Buffered(1)}
    return pl.pallas_call(
        _ffn_body,
        out_shape=jax.ShapeDtypeStruct((m_rows, d_out), out_dtype),
        grid=(m_rows // tile_m,),
        in_specs=[
            pl.BlockSpec((tile_m, d_in), lambda i: (i, 0)),
            pl.BlockSpec((d_in, d_mid), const, **wkw),
            pl.BlockSpec((1, d_mid), const, **wkw),
            pl.BlockSpec((d_mid, d_out), const, **wkw),
            pl.BlockSpec((1, d_out), const, **wkw),
        ],
        out_specs=pl.BlockSpec((tile_m, d_out), lambda i: (i, 0)),
        compiler_params=pltpu.CompilerParams(
            dimension_semantics=("parallel",),
            vmem_limit_bytes=60 * 1024 * 1024,
        ),
    )


@jax.jit
def kernel(x, w1, b1, w2, b2):
    B, S, H = x.shape
    FF = w1.shape[1]
    M = B * S
    x2 = x.reshape(M, H)

    w1b = w1.astype(jnp.bfloat16)
    w2b = w2.astype(jnp.bfloat16)
    b1f = b1.astype(jnp.float32).reshape(1, FF)
    b2f = b2.astype(jnp.float32).reshape(1, H)

    tile_m = _TILE_M
    while M % tile_m:
        tile_m //= 2

    out = _ffn_call(M, tile_m, H, FF, H, x.dtype)(x2, w1b, b1f, w2b, b2f)
    return out.reshape(B, S, H)


# speedup vs baseline: 1.0691x; 1.0691x over previous
import jax
import jax.numpy as jnp
from jax.experimental import pallas as pl
from jax.experimental.pallas import tpu as pltpu

_TILE_M = 1024


def _ffn_body(x_ref, w1_ref, b1_ref, w2_ref, b2_ref, o_ref):
    xb = x_ref[...].astype(jnp.bfloat16)
    h = jnp.dot(xb, w1_ref[...], preferred_element_type=jnp.float32)
    h = jnp.maximum(h + b1_ref[...], 0.0).astype(jnp.bfloat16)
    y = jnp.dot(h, w2_ref[...], preferred_element_type=jnp.float32)
    o_ref[...] = (y + b2_ref[...]).astype(o_ref.dtype)


def _ffn_call(m_rows, tile_m, d_in, d_mid, d_out, out_dtype):
    n_half = m_rows // tile_m // 2
    const = lambda c, j: (0, 0)
    wkw = {"pipeline_mode": pl.Buffered(1)}
    return pl.pallas_call(
        _ffn_body,
        out_shape=jax.ShapeDtypeStruct((m_rows, d_out), out_dtype),
        grid=(2, n_half),
        in_specs=[
            pl.BlockSpec((tile_m, d_in), lambda c, j: (c * n_half + j, 0)),
            pl.BlockSpec((d_in, d_mid), const, **wkw),
            pl.BlockSpec((1, d_mid), const, **wkw),
            pl.BlockSpec((d_mid, d_out), const, **wkw),
            pl.BlockSpec((1, d_out), const, **wkw),
        ],
        out_specs=pl.BlockSpec((tile_m, d_out), lambda c, j: (c * n_half + j, 0)),
        compiler_params=pltpu.CompilerParams(
            dimension_semantics=("parallel", "arbitrary"),
            vmem_limit_bytes=60 * 1024 * 1024,
        ),
    )


@jax.jit
def kernel(x, w1, b1, w2, b2):
    B, S, H = x.shape
    FF = w1.shape[1]
    M = B * S
    x2 = x.reshape(M, H)

    w1b = w1.astype(jnp.bfloat16)
    w2b = w2.astype(jnp.bfloat16)
    b1f = b1.astype(jnp.float32).reshape(1, FF)
    b2f = b2.astype(jnp.float32).reshape(1, H)

    tile_m = _TILE_M
    while M % (2 * tile_m):
        tile_m //= 2

    out = _ffn_call(M, tile_m, H, FF, H, x.dtype)(x2, w1b, b1f, w2b, b2f)
    return out.reshape(B, S, H)
